# Initial kernel scaffold; baseline (speedup 1.0000x reference)
import functools

import jax
import jax.numpy as jnp
from jax import lax
from jax.experimental import pallas as pl
from jax.experimental.pallas import tpu as pltpu
from jax.experimental.pallas import tpu_sc as plsc

D = 32
NW = 32
CH = 1280


def _emb_body(table_hbm, idx_hbm, out_hbm, idx_v, rows_v, gsem, ssem,
              *, b_per_w, nch):
    wid = lax.axis_index("s") * 2 + lax.axis_index("c")
    base = wid * b_per_w
    pltpu.sync_copy(idx_hbm.at[pl.ds(base, b_per_w)], idx_v)

    def start_gather(j, buf):
        return pltpu.async_copy(
            table_hbm.at[idx_v.at[pl.ds(j * CH, CH)]], rows_v.at[buf],
            gsem.at[buf])

    def start_store(j, buf):
        return pltpu.async_copy(
            rows_v.at[buf], out_hbm.at[pl.ds(base + j * CH, CH)],
            ssem.at[buf])

    pending_gather = [None, None]
    pending_store = [None, None]
    pending_gather[0] = start_gather(0, 0)
    for j in range(nch):
        buf = j % 2
        nbuf = (j + 1) % 2
        if j + 1 < nch:
            if pending_store[nbuf] is not None:
                pending_store[nbuf].wait()
            pending_gather[nbuf] = start_gather(j + 1, nbuf)
        pending_gather[buf].wait()
        pending_store[buf] = start_store(j, buf)
    for d in pending_store:
        if d is not None:
            d.wait()


def kernel(x, weight):
    b, l = x.shape
    n = b * l
    b_per_w = n // NW
    nch = b_per_w // CH
    idx = x.reshape(n).astype(jnp.int32)

    mesh = plsc.VectorSubcoreMesh(core_axis_name="c", subcore_axis_name="s")
    body = functools.partial(_emb_body, b_per_w=b_per_w, nch=nch)
    out = pl.kernel(
        body,
        out_type=jax.ShapeDtypeStruct((n, D), jnp.float32),
        mesh=mesh,
        scratch_types=[
            pltpu.VMEM((b_per_w,), jnp.int32),
            pltpu.VMEM((2, CH, D), jnp.float32),
            pltpu.SemaphoreType.DMA((2,)),
            pltpu.SemaphoreType.DMA((2,)),
        ],
    )(weight, idx)
    return out.reshape(b, l, D)


# speedup vs baseline: 1.1140x; 1.1140x over previous
import functools

import jax
import jax.numpy as jnp
from jax import lax
from jax.experimental import pallas as pl
from jax.experimental.pallas import tpu as pltpu
from jax.experimental.pallas import tpu_sc as plsc

D = 32
NW = 32
CH = 1280


def _emb_body(table_hbm, idx_hbm, out_hbm, idx_v, rows_v, gsem, ssem,
              *, b_per_w, nch):
    wid = lax.axis_index("s") * 2 + lax.axis_index("c")
    base = wid * b_per_w
    pltpu.sync_copy(idx_hbm.at[pl.ds(base, b_per_w)], idx_v)

    def start_gather(j, buf):
        return pltpu.async_copy(
            table_hbm.at[idx_v.at[pl.ds(j * CH, CH)]], rows_v.at[buf],
            gsem.at[buf])

    def start_store(j, buf):
        return pltpu.async_copy(
            rows_v.at[buf], out_hbm.at[pl.ds(base + j * CH, CH)],
            ssem.at[buf])

    pending_gather = [None, None]
    pending_store = [None, None]
    pending_gather[0] = start_gather(0, 0)
    for j in range(nch):
        buf = j % 2
        nbuf = (j + 1) % 2
        if j + 1 < nch:
            if pending_store[nbuf] is not None:
                pending_store[nbuf].wait()
            pending_gather[nbuf] = start_gather(j + 1, nbuf)
        pending_gather[buf].wait()
        pending_store[buf] = start_store(j, buf)
    for d in pending_store:
        if d is not None:
            d.wait()


def kernel(x, weight):
    b, l = x.shape
    n = b * l
    b_per_w = n // NW
    nch = b_per_w // CH
    idx = x.reshape(n).astype(jnp.int32)

    mesh = plsc.VectorSubcoreMesh(core_axis_name="c", subcore_axis_name="s")
    body = functools.partial(_emb_body, b_per_w=b_per_w, nch=nch)
    out = pl.kernel(
        body,
        out_type=jax.ShapeDtypeStruct((n, D), jnp.float32),
        mesh=mesh,
        scratch_types=[
            pltpu.VMEM((b_per_w,), jnp.int32),
            pltpu.VMEM((2, CH, D), jnp.float32),
            pltpu.SemaphoreType.DMA((2,)),
            pltpu.SemaphoreType.DMA((2,)),
        ],
        compiler_params=pltpu.CompilerParams(use_tc_tiling_on_sc=False),
    )(weight, idx)
    return out.reshape(b, l, D)


# speedup vs baseline: 1.1146x; 1.0005x over previous
import functools

import jax
import jax.numpy as jnp
from jax import lax
from jax.experimental import pallas as pl
from jax.experimental.pallas import tpu as pltpu
from jax.experimental.pallas import tpu_sc as plsc

D = 32
NW = 32
CH = 400
NB = 8


def _emb_body(table_hbm, idx_hbm, out_hbm, idx_v, rows_v, gsem, ssem,
              *, b_per_w, nch):
    wid = lax.axis_index("s") * 2 + lax.axis_index("c")
    base = wid * b_per_w
    pltpu.sync_copy(idx_hbm.at[pl.ds(base, b_per_w)], idx_v)

    def start_gather(j, buf):
        off = pl.multiple_of(j * CH, 8)
        pltpu.async_copy(
            table_hbm.at[idx_v.at[pl.ds(off, CH)]], rows_v.at[buf],
            gsem.at[buf])

    def wait_gather(buf):
        pltpu.make_async_copy(
            table_hbm.at[idx_v.at[pl.ds(0, CH)]], rows_v.at[buf],
            gsem.at[buf]).wait()

    def start_store(j, buf):
        off = pl.multiple_of(base + j * CH, 8)
        pltpu.async_copy(
            rows_v.at[buf], out_hbm.at[pl.ds(off, CH)], ssem.at[buf])

    def wait_store(buf):
        pltpu.make_async_copy(
            rows_v.at[buf], out_hbm.at[pl.ds(0, CH)], ssem.at[buf]).wait()

    for b in range(NB):
        start_gather(b, b)

    @pl.loop(0, nch - NB, step=NB)
    def _steady(i):
        for b in range(NB):
            j = i + b
            wait_gather(b)
            start_store(j, b)
            wait_store(b)
            start_gather(j + NB, b)

    for b in range(NB):
        wait_gather(b)
        start_store(nch - NB + b, b)
    for b in range(NB):
        wait_store(b)


def kernel(x, weight):
    b, l = x.shape
    n = b * l
    b_per_w = n // NW
    nch = b_per_w // CH
    idx = x.reshape(n).astype(jnp.int32)

    mesh = plsc.VectorSubcoreMesh(core_axis_name="c", subcore_axis_name="s")
    body = functools.partial(_emb_body, b_per_w=b_per_w, nch=nch)
    out = pl.kernel(
        body,
        out_type=jax.ShapeDtypeStruct((n, D), jnp.float32),
        mesh=mesh,
        scratch_types=[
            pltpu.VMEM((b_per_w,), jnp.int32),
            pltpu.VMEM((NB, CH, D), jnp.float32),
            pltpu.SemaphoreType.DMA((NB,)),
            pltpu.SemaphoreType.DMA((NB,)),
        ],
        compiler_params=pltpu.CompilerParams(use_tc_tiling_on_sc=False),
    )(weight, idx)
    return out.reshape(b, l, D)


# speedup vs baseline: 1.7578x; 1.5771x over previous
import functools

import jax
import jax.numpy as jnp
from jax import lax
from jax.experimental import pallas as pl
from jax.experimental.pallas import tpu as pltpu
from jax.experimental.pallas import tpu_sc as plsc

D = 32
NW = 32
LCH = 80
NB = 8


def _emb_body(table_hbm, idx_hbm, out_hbm, idx_v, rows_v, gsem, ssem,
              *, l_per_w, nch):
    wid = lax.axis_index("s") * 2 + lax.axis_index("c")
    lbase = wid * l_per_w
    pltpu.sync_copy(idx_hbm.at[:, pl.ds(lbase, l_per_w)], idx_v)

    def start_gather(j, buf):
        off = pl.multiple_of(j * LCH, 8)
        for k in range(4):
            pltpu.async_copy(
                table_hbm.at[idx_v.at[k, pl.ds(off, LCH)]],
                rows_v.at[buf, k], gsem.at[buf])

    def wait_gather(buf):
        for k in range(4):
            pltpu.make_async_copy(
                table_hbm.at[idx_v.at[k, pl.ds(0, LCH)]],
                rows_v.at[buf, k], gsem.at[buf]).wait()

    def start_store(j, buf):
        off = pl.multiple_of(lbase + j * LCH, 8)
        for k in range(4):
            pltpu.async_copy(
                rows_v.at[buf, k],
                out_hbm.at[pl.ds(off, LCH), pl.ds(k * D, D)], ssem.at[buf])

    def wait_store(buf):
        for k in range(4):
            pltpu.make_async_copy(
                rows_v.at[buf, k],
                out_hbm.at[pl.ds(0, LCH), pl.ds(k * D, D)], ssem.at[buf]).wait()

    for b in range(NB):
        start_gather(b, b)

    @pl.loop(0, nch - NB, step=NB)
    def _steady(i):
        for b in range(NB):
            j = i + b
            wait_gather(b)
            start_store(j, b)
            wait_store(b)
            start_gather(j + NB, b)

    for b in range(NB):
        wait_gather(b)
        start_store(nch - NB + b, b)
    for b in range(NB):
        wait_store(b)


def kernel(x, weight):
    b, l = x.shape
    n = b * l
    nlines = n // 4
    l_per_w = nlines // NW
    nch = l_per_w // LCH
    idx4t = x.reshape(nlines, 4).astype(jnp.int32).T

    mesh = plsc.VectorSubcoreMesh(core_axis_name="c", subcore_axis_name="s")
    body = functools.partial(_emb_body, l_per_w=l_per_w, nch=nch)
    out = pl.kernel(
        body,
        out_type=jax.ShapeDtypeStruct((nlines, 4 * D), jnp.float32),
        mesh=mesh,
        scratch_types=[
            pltpu.VMEM((4, l_per_w), jnp.int32),
            pltpu.VMEM((NB, 4, LCH, D), jnp.float32),
            pltpu.SemaphoreType.DMA((NB,)),
            pltpu.SemaphoreType.DMA((NB,)),
        ],
        compiler_params=pltpu.CompilerParams(use_tc_tiling_on_sc=False),
    )(weight, idx4t)
    return out.reshape(b, l, D)


# speedup vs baseline: 1.8111x; 1.0303x over previous
import functools

import jax
import jax.numpy as jnp
from jax import lax
from jax.experimental import pallas as pl
from jax.experimental.pallas import tpu as pltpu
from jax.experimental.pallas import tpu_sc as plsc

D = 32
NW = 32
SCH = 8
NB = 8


def _emb_body(table_hbm, idx_hbm, out_hbm, idx_v, rows_v, gsem, ssem,
              *, l, b_per_w, nch):
    ch = SCH * l
    wid = lax.axis_index("s") * 2 + lax.axis_index("c")
    base = wid * b_per_w
    sbase = wid * (b_per_w // l)
    pltpu.sync_copy(idx_hbm.at[pl.ds(base, b_per_w)], idx_v)

    def start_gather(j, buf):
        off = pl.multiple_of(j * ch, 8)
        pltpu.async_copy(
            table_hbm.at[idx_v.at[pl.ds(off, ch)]], rows_v.at[buf],
            gsem.at[buf])

    def wait_gather(buf):
        pltpu.make_async_copy(
            table_hbm.at[idx_v.at[pl.ds(0, ch)]], rows_v.at[buf],
            gsem.at[buf]).wait()

    def start_store(j, buf):
        off = sbase + j * SCH
        for s in range(SCH):
            pltpu.async_copy(
                rows_v.at[buf, pl.ds(s * l, l)], out_hbm.at[off + s],
                ssem.at[buf])

    def wait_store(buf):
        for s in range(SCH):
            pltpu.make_async_copy(
                rows_v.at[buf, pl.ds(s * l, l)], out_hbm.at[s],
                ssem.at[buf]).wait()

    for b in range(NB):
        start_gather(b, b)

    @pl.loop(0, nch - NB, step=NB)
    def _steady(i):
        for b in range(NB):
            j = i + b
            wait_gather(b)
            start_store(j, b)
            wait_store(b)
            start_gather(j + NB, b)

    for b in range(NB):
        wait_gather(b)
        start_store(nch - NB + b, b)
    for b in range(NB):
        wait_store(b)


def kernel(x, weight):
    b, l = x.shape
    n = b * l
    b_per_w = n // NW
    nch = b_per_w // (SCH * l)
    idx = x.reshape(n).astype(jnp.int32)

    mesh = plsc.VectorSubcoreMesh(core_axis_name="c", subcore_axis_name="s")
    body = functools.partial(_emb_body, l=l, b_per_w=b_per_w, nch=nch)
    out = pl.kernel(
        body,
        out_type=jax.ShapeDtypeStruct((b, l, D), jnp.float32),
        mesh=mesh,
        scratch_types=[
            pltpu.VMEM((b_per_w,), jnp.int32),
            pltpu.VMEM((NB, SCH * l, D), jnp.float32),
            pltpu.SemaphoreType.DMA((NB,)),
            pltpu.SemaphoreType.DMA((NB,)),
        ],
        compiler_params=pltpu.CompilerParams(use_tc_tiling_on_sc=False),
    )(weight, idx)
    return out
